# baseline (speedup 1.0000x reference)
import functools

import jax
import jax.numpy as jnp
from jax import lax
from jax.experimental import pallas as pl
from jax.experimental.pallas import tpu as pltpu
from jax.experimental.pallas import tpu_sc as plsc

N = 100000
E = 1600000
D_IN, D_HID, D_OUT = 1433, 32, 7
HALF = 16

NC, NS = 2, 16
NT = NC * NS
C = 784
NB2 = 64
NB1 = 2 * NB2
EPT = C * NB2
EPT1 = C * NB1
E_PAD = NT * EPT
N_DEG = 100352
ZPT = 6256
N_ACC = NS * ZPT
DPT = N_DEG // NS

BN = 2048
GRID = (N + BN - 1) // BN

_mesh = plsc.VectorSubcoreMesh(
    core_axis_name="c", subcore_axis_name="s", num_cores=NC, num_subcores=NS)
_sc_params = pltpu.CompilerParams(
    needs_layout_passes=False, use_tc_tiling_on_sc=False)


@functools.partial(
    pl.kernel,
    out_type=jax.ShapeDtypeStruct((NC, N_DEG), jnp.float32),
    mesh=_mesh,
    compiler_params=_sc_params,
    scratch_types=[
        pltpu.VMEM((C,), jnp.int32),
        pltpu.VMEM((C,), jnp.int32),
        pltpu.VMEM((C,), jnp.float32),
        pltpu.VMEM_SHARED((N_DEG,), jnp.float32),
        pltpu.SemaphoreType.DMA,
    ],
)
def _deg_sc(dst_hbm, zdeg_hbm, out_hbm, didx_a, didx_b, ones_v, dacc_sh, sem):
    c = lax.axis_index("c")
    s = lax.axis_index("s")
    base = (c * NS + s) * EPT
    pltpu.sync_copy(zdeg_hbm.at[pl.ds(s * DPT, DPT)],
                    dacc_sh.at[pl.ds(s * DPT, DPT)])

    def wr(i, carry):
        ones_v[pl.ds(i * 16, 16)] = jnp.full((16,), 1.0, jnp.float32)
        return carry

    lax.fori_loop(0, C // 16, wr, None)
    plsc.subcore_barrier()

    pltpu.sync_copy(dst_hbm.at[pl.ds(base, C)], didx_a)

    def body(t, carry):
        def chunk(cur, nxt):
            cp = pltpu.async_copy(ones_v, dacc_sh.at[cur], sem, add=True)

            @pl.when(t < NB2 - 1)
            def _():
                pltpu.sync_copy(dst_hbm.at[pl.ds(base + (t + 1) * C, C)], nxt)
            cp.wait()

        @pl.when(lax.rem(t, 2) == 0)
        def _():
            chunk(didx_a, didx_b)

        @pl.when(lax.rem(t, 2) == 1)
        def _():
            chunk(didx_b, didx_a)
        return carry

    lax.fori_loop(0, NB2, body, None)
    plsc.subcore_barrier()
    pltpu.sync_copy(dacc_sh.at[pl.ds(s * DPT, DPT)],
                    out_hbm.at[c, pl.ds(s * DPT, DPT)])


def _edge_pass(table_a, table_b, src_hbm, dst_hbm,
               sidx_a, sidx_b, didx_a, didx_b, rows_a, rows_b,
               acc_sh, gsem, ssem, base, nb):
    def run(table):
        pltpu.sync_copy(src_hbm.at[pl.ds(base, C)], sidx_a)
        pltpu.sync_copy(dst_hbm.at[pl.ds(base, C)], didx_a)
        pltpu.async_copy(table.at[sidx_a], rows_a, gsem)

        def body(t, carry):
            def chunk(si_c, di_c, ro_c, si_n, di_n, ro_n):
                pltpu.make_async_copy(table.at[si_c], ro_c, gsem).wait()

                @pl.when(t >= 1)
                def _():
                    pltpu.make_async_copy(
                        ro_n, acc_sh.at[di_n], ssem).wait()
                pltpu.async_copy(ro_c, acc_sh.at[di_c], ssem, add=True)

                @pl.when(t < nb - 1)
                def _():
                    nb_ = base + (t + 1) * C
                    pltpu.sync_copy(src_hbm.at[pl.ds(nb_, C)], si_n)
                    pltpu.sync_copy(dst_hbm.at[pl.ds(nb_, C)], di_n)
                    pltpu.async_copy(table.at[si_n], ro_n, gsem)

            @pl.when(lax.rem(t, 2) == 0)
            def _():
                chunk(sidx_a, didx_a, rows_a, sidx_b, didx_b, rows_b)

            @pl.when(lax.rem(t, 2) == 1)
            def _():
                chunk(sidx_b, didx_b, rows_b, sidx_a, didx_a, rows_a)
            return carry

        lax.fori_loop(0, nb, body, None)
        pf = (nb - 1) % 2
        if pf == 0:
            pltpu.make_async_copy(rows_a, acc_sh.at[didx_a], ssem).wait()
        else:
            pltpu.make_async_copy(rows_b, acc_sh.at[didx_b], ssem).wait()

    if table_b is None:
        run(table_a)
    else:
        c = lax.axis_index("c")

        @pl.when(c == 0)
        def _():
            run(table_a)

        @pl.when(c == 1)
        def _():
            run(table_b)


_SC_SCRATCH = [
    pltpu.VMEM((C,), jnp.int32),
    pltpu.VMEM((C,), jnp.int32),
    pltpu.VMEM((C,), jnp.int32),
    pltpu.VMEM((C,), jnp.int32),
    pltpu.VMEM((C, HALF), jnp.float32),
    pltpu.VMEM((C, HALF), jnp.float32),
    pltpu.VMEM_SHARED((N_ACC, HALF), jnp.float32),
    pltpu.SemaphoreType.DMA,
    pltpu.SemaphoreType.DMA,
]


def _zero_acc(zrows_hbm, acc_sh, s):
    pltpu.sync_copy(zrows_hbm, acc_sh.at[pl.ds(s * ZPT, ZPT)])
    plsc.subcore_barrier()


def _write_acc(out_hbm, acc_sh, c, s):
    plsc.subcore_barrier()

    @pl.when(s < NS - 1)
    def _():
        pltpu.sync_copy(acc_sh.at[pl.ds(s * ZPT, ZPT)],
                        out_hbm.at[c, pl.ds(s * ZPT, ZPT)])

    @pl.when(s == NS - 1)
    def _():
        tail = N - (NS - 1) * ZPT
        pltpu.sync_copy(acc_sh.at[pl.ds((NS - 1) * ZPT, tail)],
                        out_hbm.at[c, pl.ds((NS - 1) * ZPT, tail)])


@functools.partial(
    pl.kernel,
    out_type=jax.ShapeDtypeStruct((NC, N, HALF), jnp.float32),
    mesh=_mesh,
    compiler_params=_sc_params,
    scratch_types=_SC_SCRATCH,
)
def _gcn1_sc(ta_hbm, tb_hbm, src_hbm, dst_hbm, zrows_hbm, out_hbm,
             sidx_a, sidx_b, didx_a, didx_b, rows_a, rows_b,
             acc_sh, gsem, ssem):
    c = lax.axis_index("c")
    s = lax.axis_index("s")
    _zero_acc(zrows_hbm, acc_sh, s)
    _edge_pass(ta_hbm, tb_hbm, src_hbm, dst_hbm,
               sidx_a, sidx_b, didx_a, didx_b, rows_a, rows_b,
               acc_sh, gsem, ssem, s * EPT1, NB1)
    _write_acc(out_hbm, acc_sh, c, s)


@functools.partial(
    pl.kernel,
    out_type=jax.ShapeDtypeStruct((NC, N, HALF), jnp.float32),
    mesh=_mesh,
    compiler_params=_sc_params,
    scratch_types=_SC_SCRATCH,
)
def _gcn2_sc(t_hbm, src_hbm, dst_hbm, zrows_hbm, out_hbm,
             sidx_a, sidx_b, didx_a, didx_b, rows_a, rows_b,
             acc_sh, gsem, ssem):
    c = lax.axis_index("c")
    s = lax.axis_index("s")
    _zero_acc(zrows_hbm, acc_sh, s)
    _edge_pass(t_hbm, None, src_hbm, dst_hbm,
               sidx_a, sidx_b, didx_a, didx_b, rows_a, rows_b,
               acc_sh, gsem, ssem, (c * NS + s) * EPT, NB2)
    _write_acc(out_hbm, acc_sh, c, s)


def _dinv_body(degp_ref, dinv_ref):
    deg = jnp.sum(degp_ref[...], axis=0, keepdims=True) + 1.0
    dinv_ref[...] = lax.rsqrt(deg)


def _k1a_body(xt_ref, w1_ref, h_ref):
    xtb = xt_ref[...].astype(jnp.bfloat16)
    wb = w1_ref[...].astype(jnp.bfloat16)
    h_ref[...] = lax.dot_general(xtb, wb, (((0,), (0,)), ((), ())),
                                 preferred_element_type=jnp.float32)


def _k1b_body(h_ref, dinv_ref, g1a_ref, g1b_ref):
    g = h_ref[...] * dinv_ref[...]
    g1a_ref[...] = g[:, :HALF]
    g1b_ref[...] = g[:, HALF:]


def _k2_body(g1a_ref, g1b_ref, acc1_ref, dinv_ref, w2p_ref, b1_ref, g2_ref):
    dinv = dinv_ref[...]
    b1 = b1_ref[...]
    oa = (acc1_ref[0] + g1a_ref[...]) * dinv + b1[:, :HALF]
    ob = (acc1_ref[1] + g1b_ref[...]) * dinv + b1[:, HALF:]
    oa = jnp.maximum(oa, 0.0).astype(jnp.bfloat16)
    ob = jnp.maximum(ob, 0.0).astype(jnp.bfloat16)
    w2 = w2p_ref[...].astype(jnp.bfloat16)
    dn = (((1,), (0,)), ((), ()))
    h2 = (lax.dot_general(oa, w2[:HALF, :], dn,
                          preferred_element_type=jnp.float32) +
          lax.dot_general(ob, w2[HALF:, :], dn,
                          preferred_element_type=jnp.float32))
    g2_ref[...] = h2 * dinv


def _k3_body(g2_ref, p2_ref, dinv_ref, b2_ref, out_ref):
    t = (p2_ref[0] + p2_ref[1] + g2_ref[...]) * dinv_ref[...]
    res = t[:, :D_OUT] + b2_ref[...]
    out_ref[...] = jnp.transpose(res)


def kernel(x, edge_index, W1, b1, W2, b2):
    f32 = jnp.float32
    src = edge_index[0]
    dst = edge_index[1]
    pad = E_PAD - E
    src_p = jnp.concatenate([src, jnp.zeros((pad,), jnp.int32)])
    dst_p = jnp.concatenate([dst, jnp.full((pad,), N, jnp.int32)])
    zdeg = jnp.zeros((N_DEG,), f32)
    zrows = jnp.zeros((ZPT, HALF), f32)
    w2p = jnp.zeros((D_HID, HALF), f32).at[:, :D_OUT].set(W2)
    b1r = b1.reshape(1, D_HID)
    b2r = b2.reshape(1, D_OUT)
    xt = x.T

    degp = _deg_sc(dst_p, zdeg)

    h1 = pl.pallas_call(
        _k1a_body,
        grid=(GRID,),
        in_specs=[
            pl.BlockSpec((D_IN, BN), lambda i: (0, i)),
            pl.BlockSpec((D_IN, D_HID), lambda i: (0, 0)),
        ],
        out_specs=pl.BlockSpec((BN, D_HID), lambda i: (i, 0)),
        out_shape=jax.ShapeDtypeStruct((N, D_HID), f32),
    )(xt, W1)

    dinv_flat = pl.pallas_call(
        _dinv_body,
        out_shape=jax.ShapeDtypeStruct((1, N_DEG), f32),
    )(degp)
    dinv = dinv_flat[0, :N].reshape(N, 1)

    g1a, g1b = pl.pallas_call(
        _k1b_body,
        grid=(GRID,),
        in_specs=[
            pl.BlockSpec((BN, D_HID), lambda i: (i, 0)),
            pl.BlockSpec((BN, 1), lambda i: (i, 0)),
        ],
        out_specs=[
            pl.BlockSpec((BN, HALF), lambda i: (i, 0)),
            pl.BlockSpec((BN, HALF), lambda i: (i, 0)),
        ],
        out_shape=[
            jax.ShapeDtypeStruct((N, HALF), f32),
            jax.ShapeDtypeStruct((N, HALF), f32),
        ],
    )(h1, dinv)

    acc1 = _gcn1_sc(g1a, g1b, src_p, dst_p, zrows)

    g2 = pl.pallas_call(
        _k2_body,
        grid=(GRID,),
        in_specs=[
            pl.BlockSpec((BN, HALF), lambda i: (i, 0)),
            pl.BlockSpec((BN, HALF), lambda i: (i, 0)),
            pl.BlockSpec((NC, BN, HALF), lambda i: (0, i, 0)),
            pl.BlockSpec((BN, 1), lambda i: (i, 0)),
            pl.BlockSpec((D_HID, HALF), lambda i: (0, 0)),
            pl.BlockSpec((1, D_HID), lambda i: (0, 0)),
        ],
        out_specs=pl.BlockSpec((BN, HALF), lambda i: (i, 0)),
        out_shape=jax.ShapeDtypeStruct((N, HALF), f32),
    )(g1a, g1b, acc1, dinv, w2p, b1r)

    p2 = _gcn2_sc(g2, src_p, dst_p, zrows)

    out_t = pl.pallas_call(
        _k3_body,
        grid=(GRID,),
        in_specs=[
            pl.BlockSpec((BN, HALF), lambda i: (i, 0)),
            pl.BlockSpec((NC, BN, HALF), lambda i: (0, i, 0)),
            pl.BlockSpec((BN, 1), lambda i: (i, 0)),
            pl.BlockSpec((1, D_OUT), lambda i: (0, 0)),
        ],
        out_specs=pl.BlockSpec((D_OUT, BN), lambda i: (0, i)),
        out_shape=jax.ShapeDtypeStruct((D_OUT, N), f32),
    )(g2, p2, dinv, b2r)
    return out_t.T


# speedup vs baseline: 1.0078x; 1.0078x over previous
import functools

import jax
import jax.numpy as jnp
from jax import lax
from jax.experimental import pallas as pl
from jax.experimental.pallas import tpu as pltpu
from jax.experimental.pallas import tpu_sc as plsc

N = 100000
E = 1600000
D_IN, D_HID, D_OUT = 1433, 32, 7
HALF = 16

NC, NS = 2, 16
NT = NC * NS
C = 784
NB2 = 64
NB1 = 2 * NB2
EPT = C * NB2
EPT1 = C * NB1
E_PAD = NT * EPT
N_DEG = 100352
ZPT = 6256
N_ACC = NS * ZPT
DPT = N_DEG // NS

BN = 2048
GRID = (N + BN - 1) // BN

_mesh = plsc.VectorSubcoreMesh(
    core_axis_name="c", subcore_axis_name="s", num_cores=NC, num_subcores=NS)
_sc_params = pltpu.CompilerParams(
    needs_layout_passes=False, use_tc_tiling_on_sc=False)


@functools.partial(
    pl.kernel,
    out_type=jax.ShapeDtypeStruct((NC, N_DEG), jnp.float32),
    mesh=_mesh,
    compiler_params=_sc_params,
    scratch_types=[
        pltpu.VMEM((C,), jnp.int32),
        pltpu.VMEM((C,), jnp.int32),
        pltpu.VMEM((C,), jnp.float32),
        pltpu.VMEM_SHARED((N_DEG,), jnp.float32),
        pltpu.SemaphoreType.DMA,
    ],
)
def _deg_sc(dst_hbm, zdeg_hbm, out_hbm, didx_a, didx_b, ones_v, dacc_sh, sem):
    c = lax.axis_index("c")
    s = lax.axis_index("s")
    base = (c * NS + s) * EPT
    pltpu.sync_copy(zdeg_hbm.at[pl.ds(s * DPT, DPT)],
                    dacc_sh.at[pl.ds(s * DPT, DPT)])

    def wr(i, carry):
        ones_v[pl.ds(i * 16, 16)] = jnp.full((16,), 1.0, jnp.float32)
        return carry

    lax.fori_loop(0, C // 16, wr, None)
    plsc.subcore_barrier()

    pltpu.sync_copy(dst_hbm.at[pl.ds(base, C)], didx_a)

    def body(t, carry):
        def chunk(cur, nxt):
            cp = pltpu.async_copy(ones_v, dacc_sh.at[cur], sem, add=True)

            @pl.when(t < NB2 - 1)
            def _():
                pltpu.sync_copy(dst_hbm.at[pl.ds(base + (t + 1) * C, C)], nxt)
            cp.wait()

        @pl.when(lax.rem(t, 2) == 0)
        def _():
            chunk(didx_a, didx_b)

        @pl.when(lax.rem(t, 2) == 1)
        def _():
            chunk(didx_b, didx_a)
        return carry

    lax.fori_loop(0, NB2, body, None)
    plsc.subcore_barrier()
    pltpu.sync_copy(dacc_sh.at[pl.ds(s * DPT, DPT)],
                    out_hbm.at[c, pl.ds(s * DPT, DPT)])


def _edge_pass(table_a, table_b, src_hbm, dst_hbm,
               sidx_a, sidx_b, didx_a, didx_b, rows_a, rows_b,
               acc_sh, gsem, ssem, base, nb):
    def run(table):
        pltpu.sync_copy(src_hbm.at[pl.ds(base, C)], sidx_a)
        pltpu.sync_copy(dst_hbm.at[pl.ds(base, C)], didx_a)
        pltpu.async_copy(table.at[sidx_a], rows_a, gsem)

        def body(t, carry):
            def chunk(si_c, di_c, ro_c, si_n, di_n, ro_n):
                pltpu.make_async_copy(table.at[si_c], ro_c, gsem).wait()

                @pl.when(t >= 1)
                def _():
                    pltpu.make_async_copy(
                        ro_n, acc_sh.at[di_n], ssem).wait()

                @pl.when(t < nb - 1)
                def _():
                    nb_ = base + (t + 1) * C
                    pltpu.sync_copy(src_hbm.at[pl.ds(nb_, C)], si_n)
                    pltpu.sync_copy(dst_hbm.at[pl.ds(nb_, C)], di_n)
                    pltpu.async_copy(table.at[si_n], ro_n, gsem)
                pltpu.async_copy(ro_c, acc_sh.at[di_c], ssem, add=True)

            @pl.when(lax.rem(t, 2) == 0)
            def _():
                chunk(sidx_a, didx_a, rows_a, sidx_b, didx_b, rows_b)

            @pl.when(lax.rem(t, 2) == 1)
            def _():
                chunk(sidx_b, didx_b, rows_b, sidx_a, didx_a, rows_a)
            return carry

        lax.fori_loop(0, nb, body, None)
        pf = (nb - 1) % 2
        if pf == 0:
            pltpu.make_async_copy(rows_a, acc_sh.at[didx_a], ssem).wait()
        else:
            pltpu.make_async_copy(rows_b, acc_sh.at[didx_b], ssem).wait()

    if table_b is None:
        run(table_a)
    else:
        c = lax.axis_index("c")

        @pl.when(c == 0)
        def _():
            run(table_a)

        @pl.when(c == 1)
        def _():
            run(table_b)


_SC_SCRATCH = [
    pltpu.VMEM((C,), jnp.int32),
    pltpu.VMEM((C,), jnp.int32),
    pltpu.VMEM((C,), jnp.int32),
    pltpu.VMEM((C,), jnp.int32),
    pltpu.VMEM((C, HALF), jnp.float32),
    pltpu.VMEM((C, HALF), jnp.float32),
    pltpu.VMEM_SHARED((N_ACC, HALF), jnp.float32),
    pltpu.SemaphoreType.DMA,
    pltpu.SemaphoreType.DMA,
]


def _zero_acc(zrows_hbm, acc_sh, s):
    pltpu.sync_copy(zrows_hbm, acc_sh.at[pl.ds(s * ZPT, ZPT)])
    plsc.subcore_barrier()


def _write_acc(out_hbm, acc_sh, c, s):
    plsc.subcore_barrier()

    @pl.when(s < NS - 1)
    def _():
        pltpu.sync_copy(acc_sh.at[pl.ds(s * ZPT, ZPT)],
                        out_hbm.at[c, pl.ds(s * ZPT, ZPT)])

    @pl.when(s == NS - 1)
    def _():
        tail = N - (NS - 1) * ZPT
        pltpu.sync_copy(acc_sh.at[pl.ds((NS - 1) * ZPT, tail)],
                        out_hbm.at[c, pl.ds((NS - 1) * ZPT, tail)])


@functools.partial(
    pl.kernel,
    out_type=jax.ShapeDtypeStruct((NC, N, HALF), jnp.float32),
    mesh=_mesh,
    compiler_params=_sc_params,
    scratch_types=_SC_SCRATCH,
)
def _gcn1_sc(ta_hbm, tb_hbm, src_hbm, dst_hbm, zrows_hbm, out_hbm,
             sidx_a, sidx_b, didx_a, didx_b, rows_a, rows_b,
             acc_sh, gsem, ssem):
    c = lax.axis_index("c")
    s = lax.axis_index("s")
    _zero_acc(zrows_hbm, acc_sh, s)
    _edge_pass(ta_hbm, tb_hbm, src_hbm, dst_hbm,
               sidx_a, sidx_b, didx_a, didx_b, rows_a, rows_b,
               acc_sh, gsem, ssem, s * EPT1, NB1)
    _write_acc(out_hbm, acc_sh, c, s)


@functools.partial(
    pl.kernel,
    out_type=jax.ShapeDtypeStruct((NC, N, HALF), jnp.float32),
    mesh=_mesh,
    compiler_params=_sc_params,
    scratch_types=_SC_SCRATCH,
)
def _gcn2_sc(t_hbm, src_hbm, dst_hbm, zrows_hbm, out_hbm,
             sidx_a, sidx_b, didx_a, didx_b, rows_a, rows_b,
             acc_sh, gsem, ssem):
    c = lax.axis_index("c")
    s = lax.axis_index("s")
    _zero_acc(zrows_hbm, acc_sh, s)
    _edge_pass(t_hbm, None, src_hbm, dst_hbm,
               sidx_a, sidx_b, didx_a, didx_b, rows_a, rows_b,
               acc_sh, gsem, ssem, (c * NS + s) * EPT, NB2)
    _write_acc(out_hbm, acc_sh, c, s)


def _dinv_body(degp_ref, dinv_ref):
    deg = jnp.sum(degp_ref[...], axis=0, keepdims=True) + 1.0
    dinv_ref[...] = lax.rsqrt(deg)


def _k1_body(xt_ref, w1_ref, dinv_ref, g1a_ref, g1b_ref):
    xtb = xt_ref[...].astype(jnp.bfloat16)
    wb = w1_ref[...].astype(jnp.bfloat16)
    h = lax.dot_general(xtb, wb, (((0,), (0,)), ((), ())),
                        preferred_element_type=jnp.float32)
    g = h * dinv_ref[...]
    g1a_ref[...] = g[:, :HALF]
    g1b_ref[...] = g[:, HALF:]


def _k2_body(g1a_ref, g1b_ref, acc1_ref, dinv_ref, w2p_ref, b1_ref, g2_ref):
    dinv = dinv_ref[...]
    b1 = b1_ref[...]
    oa = (acc1_ref[0] + g1a_ref[...]) * dinv + b1[:, :HALF]
    ob = (acc1_ref[1] + g1b_ref[...]) * dinv + b1[:, HALF:]
    oa = jnp.maximum(oa, 0.0).astype(jnp.bfloat16)
    ob = jnp.maximum(ob, 0.0).astype(jnp.bfloat16)
    w2 = w2p_ref[...].astype(jnp.bfloat16)
    dn = (((1,), (0,)), ((), ()))
    h2 = (lax.dot_general(oa, w2[:HALF, :], dn,
                          preferred_element_type=jnp.float32) +
          lax.dot_general(ob, w2[HALF:, :], dn,
                          preferred_element_type=jnp.float32))
    g2_ref[...] = h2 * dinv


def _k3_body(g2_ref, p2_ref, dinv_ref, b2_ref, out_ref):
    t = (p2_ref[0] + p2_ref[1] + g2_ref[...]) * dinv_ref[...]
    res = t[:, :D_OUT] + b2_ref[...]
    out_ref[...] = jnp.transpose(res)


def kernel(x, edge_index, W1, b1, W2, b2):
    f32 = jnp.float32
    src = edge_index[0]
    dst = edge_index[1]
    pad = E_PAD - E
    src_p = jnp.concatenate([src, jnp.zeros((pad,), jnp.int32)])
    dst_p = jnp.concatenate([dst, jnp.full((pad,), N, jnp.int32)])
    zdeg = jnp.zeros((N_DEG,), f32)
    zrows = jnp.zeros((ZPT, HALF), f32)
    w2p = jnp.zeros((D_HID, HALF), f32).at[:, :D_OUT].set(W2)
    b1r = b1.reshape(1, D_HID)
    b2r = b2.reshape(1, D_OUT)
    xt = x.T

    degp = _deg_sc(dst_p, zdeg)

    dinv_flat = pl.pallas_call(
        _dinv_body,
        out_shape=jax.ShapeDtypeStruct((1, N_DEG), f32),
    )(degp)
    dinv = dinv_flat[0, :N].reshape(N, 1)

    g1a, g1b = pl.pallas_call(
        _k1_body,
        grid=(GRID,),
        in_specs=[
            pl.BlockSpec((D_IN, BN), lambda i: (0, i)),
            pl.BlockSpec((D_IN, D_HID), lambda i: (0, 0)),
            pl.BlockSpec((BN, 1), lambda i: (i, 0)),
        ],
        out_specs=[
            pl.BlockSpec((BN, HALF), lambda i: (i, 0)),
            pl.BlockSpec((BN, HALF), lambda i: (i, 0)),
        ],
        out_shape=[
            jax.ShapeDtypeStruct((N, HALF), f32),
            jax.ShapeDtypeStruct((N, HALF), f32),
        ],
    )(xt, W1, dinv)

    acc1 = _gcn1_sc(g1a, g1b, src_p, dst_p, zrows)

    g2 = pl.pallas_call(
        _k2_body,
        grid=(GRID,),
        in_specs=[
            pl.BlockSpec((BN, HALF), lambda i: (i, 0)),
            pl.BlockSpec((BN, HALF), lambda i: (i, 0)),
            pl.BlockSpec((NC, BN, HALF), lambda i: (0, i, 0)),
            pl.BlockSpec((BN, 1), lambda i: (i, 0)),
            pl.BlockSpec((D_HID, HALF), lambda i: (0, 0)),
            pl.BlockSpec((1, D_HID), lambda i: (0, 0)),
        ],
        out_specs=pl.BlockSpec((BN, HALF), lambda i: (i, 0)),
        out_shape=jax.ShapeDtypeStruct((N, HALF), f32),
    )(g1a, g1b, acc1, dinv, w2p, b1r)

    p2 = _gcn2_sc(g2, src_p, dst_p, zrows)

    out_t = pl.pallas_call(
        _k3_body,
        grid=(GRID,),
        in_specs=[
            pl.BlockSpec((BN, HALF), lambda i: (i, 0)),
            pl.BlockSpec((NC, BN, HALF), lambda i: (0, i, 0)),
            pl.BlockSpec((BN, 1), lambda i: (i, 0)),
            pl.BlockSpec((1, D_OUT), lambda i: (0, 0)),
        ],
        out_specs=pl.BlockSpec((D_OUT, BN), lambda i: (0, i)),
        out_shape=jax.ShapeDtypeStruct((D_OUT, N), f32),
    )(g2, p2, dinv, b2r)
    return out_t.T
